# baseline (speedup 1.0000x reference)
import jax
import jax.numpy as jnp
from jax import lax
from jax.experimental import pallas as pl
from jax.experimental.pallas import tpu as pltpu
from jax.experimental.pallas import tpu_sc as plsc

N = 10000
E = 320000
F = 128
H = 128
B = 10
L = 200

NC, NS = 2, 16
NW = NC * NS
EPW = E // NW
CH = 125
NCHUNK = EPW // CH
NPAD = 10240
RPT = NPAD // NS

_MESH = plsc.VectorSubcoreMesh(
    core_axis_name="c", subcore_axis_name="s", num_cores=NC, num_subcores=NS)


def _deg_body(src_hbm, dst_hbm, out_hbm, dego_sh, degi_sh, src_v, dst_v,
              ones_v, zb_v, sema, semb):
    c = lax.axis_index("c")
    s = lax.axis_index("s")
    wid = c * NS + s

    pltpu.sync_copy(src_hbm.at[wid], src_v)
    pltpu.sync_copy(dst_hbm.at[wid], dst_v)

    def fill(i, carry):
        zb_v[pl.ds(i * 16, 16)] = jnp.zeros((16,), jnp.float32)
        return carry

    lax.fori_loop(0, RPT // 16, fill, 0)
    for j in range(128 // 16):
        ones_v[pl.ds(j * 16, 16)] = jnp.ones((16,), jnp.float32)

    pltpu.sync_copy(zb_v, dego_sh.at[pl.ds(s * RPT, RPT)])
    pltpu.sync_copy(zb_v, degi_sh.at[pl.ds(s * RPT, RPT)])
    plsc.subcore_barrier()

    ones = ones_v.at[pl.ds(0, CH)]

    def step(i, carry):
        pltpu.async_copy(ones, dego_sh.at[src_v.at[i]], sema, add=True)
        pltpu.async_copy(ones, degi_sh.at[dst_v.at[i]], semb, add=True)
        pltpu.make_async_copy(ones, dego_sh.at[src_v.at[i]], sema).wait()
        pltpu.make_async_copy(ones, degi_sh.at[dst_v.at[i]], semb).wait()
        return carry

    lax.fori_loop(0, NCHUNK, step, 0)
    plsc.subcore_barrier()
    pltpu.sync_copy(dego_sh.at[pl.ds(s * RPT, RPT)],
                    out_hbm.at[c, 0, pl.ds(s * RPT, RPT)])
    pltpu.sync_copy(degi_sh.at[pl.ds(s * RPT, RPT)],
                    out_hbm.at[c, 1, pl.ds(s * RPT, RPT)])


_sc_deg = pl.kernel(
    _deg_body,
    out_type=jax.ShapeDtypeStruct((NC, 2, NPAD), jnp.float32),
    mesh=_MESH,
    scratch_types=[
        pltpu.VMEM_SHARED((NPAD,), jnp.float32),
        pltpu.VMEM_SHARED((NPAD,), jnp.float32),
        pltpu.VMEM((NCHUNK, CH), jnp.int32),
        pltpu.VMEM((NCHUNK, CH), jnp.int32),
        pltpu.VMEM((128,), jnp.float32),
        pltpu.VMEM((RPT,), jnp.float32),
        pltpu.SemaphoreType.DMA,
        pltpu.SemaphoreType.DMA,
    ],
)


def _agg_body(hs_hbm, src_hbm, dst_hbm, out_hbm, agg_sh, srca, dsta, srcb,
              dstb, rows_a, rows_b, isema, isemb, dsema, dsemb, gsema,
              gsemb, ssema, ssemb, osem):
    c = lax.axis_index("c")
    s = lax.axis_index("s")
    wid = c * NS + s

    def fillrow(i, carry):
        for j in range(H // 16):
            rows_a[i, pl.ds(j * 16, 16)] = jnp.zeros((16,), jnp.float32)
        return carry

    lax.fori_loop(0, CH, fillrow, 0)
    nfull = RPT // CH
    rem = RPT - nfull * CH
    for j in range(nfull):
        pltpu.async_copy(rows_a, agg_sh.at[pl.ds(s * RPT + j * CH, CH)],
                         osem)
    pltpu.async_copy(rows_a.at[pl.ds(0, rem)],
                     agg_sh.at[pl.ds(s * RPT + nfull * CH, rem)], osem)
    for j in range(nfull):
        pltpu.make_async_copy(
            rows_a, agg_sh.at[pl.ds(s * RPT + j * CH, CH)], osem).wait()
    pltpu.make_async_copy(
        rows_a.at[pl.ds(0, rem)],
        agg_sh.at[pl.ds(s * RPT + nfull * CH, rem)], osem).wait()
    plsc.subcore_barrier()

    def aload(arr, i, buf, sem):
        pltpu.async_copy(arr.at[wid, i], buf, sem)

    def await_(buf, sem):
        pltpu.make_async_copy(src_hbm.at[wid, 0], buf, sem).wait()

    def gstart(sv, buf, sem):
        pltpu.async_copy(hs_hbm.at[sv], buf, sem)

    def gwait(buf, sem):
        pltpu.make_async_copy(hs_hbm.at[srca], buf, sem).wait()

    aload(src_hbm, 0, srca, isema)
    aload(dst_hbm, 0, dsta, dsema)
    aload(src_hbm, 1, srcb, isemb)
    aload(dst_hbm, 1, dstb, dsemb)
    await_(srca, isema)
    gstart(srca, rows_a, gsema)
    await_(srcb, isemb)
    gstart(srcb, rows_b, gsemb)

    def scstart(buf, dv, ssem):
        pltpu.async_copy(buf, agg_sh.at[dv], ssem, add=True)

    def scwait(buf, dv, ssem):
        pltpu.make_async_copy(buf, agg_sh.at[dv], ssem).wait()

    def step(i2, carry):
        c0 = 2 * i2
        gwait(rows_a, gsema)

        @pl.when(c0 + 2 < NCHUNK)
        def _():
            aload(src_hbm, c0 + 2, srca, isema)

        await_(dsta, dsema)
        scstart(rows_a, dsta, ssema)
        gwait(rows_b, gsemb)

        @pl.when(c0 + 3 < NCHUNK)
        def _():
            aload(src_hbm, c0 + 3, srcb, isemb)

        await_(dstb, dsemb)
        scstart(rows_b, dstb, ssemb)

        scwait(rows_a, dsta, ssema)

        @pl.when(c0 + 2 < NCHUNK)
        def _():
            aload(dst_hbm, c0 + 2, dsta, dsema)
            await_(srca, isema)
            gstart(srca, rows_a, gsema)

        scwait(rows_b, dstb, ssemb)

        @pl.when(c0 + 3 < NCHUNK)
        def _():
            aload(dst_hbm, c0 + 3, dstb, dsemb)
            await_(srcb, isemb)
            gstart(srcb, rows_b, gsemb)

        return carry

    lax.fori_loop(0, NCHUNK // 2, step, 0)
    plsc.subcore_barrier()
    for j in range(RPT // 128):
        r0 = s * RPT + j * 128
        pltpu.async_copy(agg_sh.at[pl.ds(r0, 128)],
                         out_hbm.at[c, pl.ds(r0, 128)], osem)
    for j in range(RPT // 128):
        r0 = s * RPT + j * 128
        pltpu.make_async_copy(agg_sh.at[pl.ds(r0, 128)],
                              out_hbm.at[c, pl.ds(r0, 128)], osem).wait()


_sc_agg = pl.kernel(
    _agg_body,
    out_type=jax.ShapeDtypeStruct((NC, NPAD, H), jnp.float32),
    mesh=_MESH,
    scratch_types=[
        pltpu.VMEM_SHARED((NPAD, H), jnp.float32),
        pltpu.VMEM((CH,), jnp.int32),
        pltpu.VMEM((CH,), jnp.int32),
        pltpu.VMEM((CH,), jnp.int32),
        pltpu.VMEM((CH,), jnp.int32),
        pltpu.VMEM((CH, H), jnp.float32),
        pltpu.VMEM((CH, H), jnp.float32),
        pltpu.SemaphoreType.DMA,
        pltpu.SemaphoreType.DMA,
        pltpu.SemaphoreType.DMA,
        pltpu.SemaphoreType.DMA,
        pltpu.SemaphoreType.DMA,
        pltpu.SemaphoreType.DMA,
        pltpu.SemaphoreType.DMA,
        pltpu.SemaphoreType.DMA,
        pltpu.SemaphoreType.DMA,
    ],
)


BLK = 1000
_NEG = -3.0e38


def _prep_body(x_ref, w_ref, deg_ref, gid_ref, hs_ref, ns_ref, nd_ref,
               cnt_ref):
    d = deg_ref[...]
    ns = lax.rsqrt(jnp.maximum(d[0, 0] + d[1, 0], 1.0))
    nd = lax.rsqrt(jnp.maximum(d[0, 1] + d[1, 1], 1.0))
    ns_ref[...] = ns
    nd_ref[...] = nd
    h = jnp.dot(x_ref[...], w_ref[...], preferred_element_type=jnp.float32)
    hs_ref[...] = h * ns
    mask = (gid_ref[...] == lax.broadcasted_iota(jnp.int32, (1, 16), 1))
    maskf = mask.astype(jnp.float32)
    cpart = lax.dot_general(maskf, jnp.ones((BLK, 1), jnp.float32),
                            (((0,), (0,)), ((), ())))

    @pl.when(pl.program_id(0) == 0)
    def _():
        cnt_ref[...] = jnp.zeros_like(cnt_ref)

    cnt_ref[...] += cpart


_tc_prep = pl.pallas_call(
    _prep_body,
    grid=(N // BLK,),
    in_specs=[
        pl.BlockSpec((BLK, F), lambda i: (i, 0)),
        pl.BlockSpec((F, H), lambda i: (0, 0)),
        pl.BlockSpec((2, 2, BLK, 1), lambda i: (0, 0, i, 0)),
        pl.BlockSpec((BLK, 1), lambda i: (i, 0)),
    ],
    out_specs=[
        pl.BlockSpec((BLK, H), lambda i: (i, 0)),
        pl.BlockSpec((BLK, 1), lambda i: (i, 0)),
        pl.BlockSpec((BLK, 1), lambda i: (i, 0)),
        pl.BlockSpec((16, 1), lambda i: (0, 0)),
    ],
    out_shape=[
        jax.ShapeDtypeStruct((N, H), jnp.float32),
        jax.ShapeDtypeStruct((N, 1), jnp.float32),
        jax.ShapeDtypeStruct((N, 1), jnp.float32),
        jax.ShapeDtypeStruct((16, 1), jnp.float32),
    ],
)


def _lupool(agg_ref, nd_ref, b_ref, p_ref):
    a = agg_ref[0] + agg_ref[1]
    g = jnp.maximum(a * nd_ref[...] + b_ref[...], 0.0)
    p = p_ref[...]
    scale = 1.0 / (jnp.sqrt(jnp.sum(p * p)) + 1e-8)
    score = lax.dot_general(g, p, (((1,), (1,)), ((), ())))
    return g * jnp.tanh(score * scale)


def _readout(xl, gid, av_ref, mx_ref):
    mask = (gid == lax.broadcasted_iota(jnp.int32, (1, 16), 1))
    maskf = mask.astype(jnp.float32)
    avp = lax.dot_general(maskf, xl, (((0,), (0,)), ((), ())))

    @pl.when(pl.program_id(0) == 0)
    def _():
        av_ref[...] = jnp.zeros_like(av_ref)
        mx_ref[...] = jnp.full_like(mx_ref, _NEG)

    av_ref[...] += avp
    rows = [jnp.max(jnp.where(mask[:, b:b + 1], xl, _NEG), axis=0,
                    keepdims=True) for b in range(16)]
    mx_ref[...] = jnp.maximum(mx_ref[...], jnp.concatenate(rows, axis=0))


def _post_body(agg_ref, nd_ref, b_ref, p_ref, w_ref, ns_ref, gid_ref,
               hs_ref, av_ref, mx_ref):
    xl = _lupool(agg_ref, nd_ref, b_ref, p_ref)
    hs_ref[...] = jnp.dot(xl, w_ref[...],
                          preferred_element_type=jnp.float32) * ns_ref[...]
    _readout(xl, gid_ref[...], av_ref, mx_ref)


_tc_post = pl.pallas_call(
    _post_body,
    grid=(N // BLK,),
    in_specs=[
        pl.BlockSpec((NC, BLK, H), lambda i: (0, i, 0)),
        pl.BlockSpec((BLK, 1), lambda i: (i, 0)),
        pl.BlockSpec((1, H), lambda i: (0, 0)),
        pl.BlockSpec((1, H), lambda i: (0, 0)),
        pl.BlockSpec((H, H), lambda i: (0, 0)),
        pl.BlockSpec((BLK, 1), lambda i: (i, 0)),
        pl.BlockSpec((BLK, 1), lambda i: (i, 0)),
    ],
    out_specs=[
        pl.BlockSpec((BLK, H), lambda i: (i, 0)),
        pl.BlockSpec((16, H), lambda i: (0, 0)),
        pl.BlockSpec((16, H), lambda i: (0, 0)),
    ],
    out_shape=[
        jax.ShapeDtypeStruct((N, H), jnp.float32),
        jax.ShapeDtypeStruct((16, H), jnp.float32),
        jax.ShapeDtypeStruct((16, H), jnp.float32),
    ],
)


def _post3_body(agg_ref, nd_ref, b_ref, p_ref, gid_ref, av_ref, mx_ref):
    xl = _lupool(agg_ref, nd_ref, b_ref, p_ref)
    _readout(xl, gid_ref[...], av_ref, mx_ref)


_tc_post3 = pl.pallas_call(
    _post3_body,
    grid=(N // BLK,),
    in_specs=[
        pl.BlockSpec((NC, BLK, H), lambda i: (0, i, 0)),
        pl.BlockSpec((BLK, 1), lambda i: (i, 0)),
        pl.BlockSpec((1, H), lambda i: (0, 0)),
        pl.BlockSpec((1, H), lambda i: (0, 0)),
        pl.BlockSpec((BLK, 1), lambda i: (i, 0)),
    ],
    out_specs=[
        pl.BlockSpec((16, H), lambda i: (0, 0)),
        pl.BlockSpec((16, H), lambda i: (0, 0)),
    ],
    out_shape=[
        jax.ShapeDtypeStruct((16, H), jnp.float32),
        jax.ShapeDtypeStruct((16, H), jnp.float32),
    ],
)


def _final_body(av1, av2, av3, mx1, mx2, mx3, cnt, xm1, x0, xp1, wc, cb,
                wtf, btf, wcat, bcat, wfc1, bfc1, wfc2, bfc2, wout, bout,
                w1, out_ref):
    counts = cnt[...]
    have = counts > 0.0
    c = jnp.maximum(counts, 1.0)
    av = (av1[...] + av2[...] + av3[...]) / c
    mx = (jnp.where(have, mx1[...], 0.0) + jnp.where(have, mx2[...], 0.0)
          + jnp.where(have, mx3[...], 0.0))
    rsum = jnp.concatenate([av, mx], axis=1)
    gnn = jnp.maximum(
        jnp.dot(rsum, wcat[...], preferred_element_type=jnp.float32)
        + bcat[...], 0.0)[0:B]

    conv = (jnp.dot(xm1[...], wc[0], preferred_element_type=jnp.float32)
            + jnp.dot(x0[...], wc[1], preferred_element_type=jnp.float32)
            + jnp.dot(xp1[...], wc[2], preferred_element_type=jnp.float32)
            + cb[...])
    conv = jnp.maximum(conv, 0.0)
    seq = jnp.max(conv.reshape(B, L, 128), axis=1)
    seq1 = jnp.maximum(
        jnp.dot(seq, wtf[...], preferred_element_type=jnp.float32)
        + btf[...], 0.0)

    s = jax.nn.sigmoid(w1[0, 0])
    gc1 = (1.0 - s) * gnn + s * seq1
    gc = jnp.maximum(
        jnp.dot(gc1, wfc1[...], preferred_element_type=jnp.float32)
        + bfc1[...], 0.0)
    gc = jnp.maximum(
        jnp.dot(gc, wfc2[...], preferred_element_type=jnp.float32)
        + bfc2[...], 0.0)
    o = jnp.maximum(
        jnp.dot(gc, wout[...], preferred_element_type=jnp.float32)
        + bout[...], 0.0)
    m = jnp.max(o, axis=1, keepdims=True)
    e = jnp.exp(o - m)
    out_ref[...] = e / jnp.sum(e, axis=1, keepdims=True)


_tc_final = pl.pallas_call(
    _final_body,
    out_shape=jax.ShapeDtypeStruct((B, 2), jnp.float32),
)


def kernel(x, edge_index, graph_ids, pad_dmap, Wg1, bg1, p1, Wg2, bg2, p2,
           Wg3, bg3, p3, conv_w, conv_b, Wtf, btf, Wcat, bcat, Wfc1, bfc1,
           Wfc2, bfc2, Wout, bout, w1):
    src = edge_index[0].reshape(NW, NCHUNK, CH)
    dst = edge_index[1].reshape(NW, NCHUNK, CH)
    gid2 = graph_ids.reshape(N, 1)

    deg = _sc_deg(src, dst).reshape(NC, 2, NPAD, 1)
    hs1, ns, nd, counts = _tc_prep(x, Wg1, deg, gid2)
    a1 = _sc_agg(hs1, src, dst)
    hs2, av1, mx1 = _tc_post(a1, nd, bg1.reshape(1, H), p1.reshape(1, H),
                             Wg2, ns, gid2)
    a2 = _sc_agg(hs2, src, dst)
    hs3, av2, mx2 = _tc_post(a2, nd, bg2.reshape(1, H), p2.reshape(1, H),
                             Wg3, ns, gid2)
    a3 = _sc_agg(hs3, src, dst)
    av3, mx3 = _tc_post3(a3, nd, bg3.reshape(1, H), p3.reshape(1, H), gid2)

    xt = jnp.transpose(pad_dmap, (0, 2, 1))
    xm1 = jnp.pad(xt, ((0, 0), (1, 0), (0, 0)))[:, :L].reshape(B * L, F)
    x0 = xt.reshape(B * L, F)
    xp1 = jnp.pad(xt, ((0, 0), (0, 1), (0, 0)))[:, 1:].reshape(B * L, F)
    wc = jnp.stack([conv_w[:, :, k].T for k in range(3)])

    return _tc_final(av1, av2, av3, mx1, mx2, mx3, counts, xm1, x0, xp1,
                     wc, conv_b.reshape(1, -1), Wtf, btf.reshape(1, -1),
                     Wcat, bcat.reshape(1, -1), Wfc1, bfc1.reshape(1, -1),
                     Wfc2, bfc2.reshape(1, -1), Wout, bout.reshape(1, -1),
                     w1.reshape(1, 1))


# speedup vs baseline: 1.3067x; 1.3067x over previous
import jax
import jax.numpy as jnp
from jax import lax
from jax.experimental import pallas as pl
from jax.experimental.pallas import tpu as pltpu
from jax.experimental.pallas import tpu_sc as plsc

N = 10000
E = 320000
F = 128
H = 128
B = 10
L = 200

NC, NS = 2, 16
NW = NC * NS
EPW = E // NW
CH = 125
NCHUNK = EPW // CH
NPAD = 10240
RPT = NPAD // NS

_MESH = plsc.VectorSubcoreMesh(
    core_axis_name="c", subcore_axis_name="s", num_cores=NC, num_subcores=NS)


def _deg_body(src_hbm, dst_hbm, out_hbm, dego_sh, degi_sh, src_v, dst_v,
              ones_v, zb_v, sema, semb):
    c = lax.axis_index("c")
    s = lax.axis_index("s")
    wid = c * NS + s

    pltpu.sync_copy(src_hbm.at[wid], src_v)
    pltpu.sync_copy(dst_hbm.at[wid], dst_v)

    def fill(i, carry):
        zb_v[pl.ds(i * 16, 16)] = jnp.zeros((16,), jnp.float32)
        return carry

    lax.fori_loop(0, RPT // 16, fill, 0)
    for j in range(128 // 16):
        ones_v[pl.ds(j * 16, 16)] = jnp.ones((16,), jnp.float32)

    pltpu.sync_copy(zb_v, dego_sh.at[pl.ds(s * RPT, RPT)])
    pltpu.sync_copy(zb_v, degi_sh.at[pl.ds(s * RPT, RPT)])
    plsc.subcore_barrier()

    ones = ones_v.at[pl.ds(0, CH)]

    def step(i, carry):
        pltpu.async_copy(ones, dego_sh.at[src_v.at[i]], sema, add=True)
        pltpu.async_copy(ones, degi_sh.at[dst_v.at[i]], semb, add=True)
        pltpu.make_async_copy(ones, dego_sh.at[src_v.at[i]], sema).wait()
        pltpu.make_async_copy(ones, degi_sh.at[dst_v.at[i]], semb).wait()
        return carry

    lax.fori_loop(0, NCHUNK, step, 0)
    plsc.subcore_barrier()
    pltpu.sync_copy(dego_sh.at[pl.ds(s * RPT, RPT)],
                    out_hbm.at[c, 0, pl.ds(s * RPT, RPT)])
    pltpu.sync_copy(degi_sh.at[pl.ds(s * RPT, RPT)],
                    out_hbm.at[c, 1, pl.ds(s * RPT, RPT)])


_sc_deg = pl.kernel(
    _deg_body,
    out_type=jax.ShapeDtypeStruct((NC, 2, NPAD), jnp.float32),
    mesh=_MESH,
    scratch_types=[
        pltpu.VMEM_SHARED((NPAD,), jnp.float32),
        pltpu.VMEM_SHARED((NPAD,), jnp.float32),
        pltpu.VMEM((NCHUNK, CH), jnp.int32),
        pltpu.VMEM((NCHUNK, CH), jnp.int32),
        pltpu.VMEM((128,), jnp.float32),
        pltpu.VMEM((RPT,), jnp.float32),
        pltpu.SemaphoreType.DMA,
        pltpu.SemaphoreType.DMA,
    ],
)


def _agg_body(hs_hbm, src_hbm, dst_hbm, out_hbm, agg_sh, srca, dsta, srcb,
              dstb, rows_a, rows_b, isema, isemb, dsema, dsemb, gsema,
              gsemb, ssema, ssemb, osem):
    c = lax.axis_index("c")
    s = lax.axis_index("s")
    wid = c * NS + s

    def fillrow(i, carry):
        for j in range(H // 16):
            rows_a[i, pl.ds(j * 16, 16)] = jnp.zeros((16,), jnp.float32)
        return carry

    lax.fori_loop(0, CH, fillrow, 0)
    nfull = RPT // CH
    rem = RPT - nfull * CH
    for j in range(nfull):
        pltpu.async_copy(rows_a, agg_sh.at[pl.ds(s * RPT + j * CH, CH)],
                         osem)
    pltpu.async_copy(rows_a.at[pl.ds(0, rem)],
                     agg_sh.at[pl.ds(s * RPT + nfull * CH, rem)], osem)
    for j in range(nfull):
        pltpu.make_async_copy(
            rows_a, agg_sh.at[pl.ds(s * RPT + j * CH, CH)], osem).wait()
    pltpu.make_async_copy(
        rows_a.at[pl.ds(0, rem)],
        agg_sh.at[pl.ds(s * RPT + nfull * CH, rem)], osem).wait()
    plsc.subcore_barrier()

    def aload(arr, i, buf, sem):
        pltpu.async_copy(arr.at[wid, i], buf, sem)

    def await_(buf, sem):
        pltpu.make_async_copy(src_hbm.at[wid, 0], buf, sem).wait()

    def gstart(sv, buf, sem):
        pltpu.async_copy(hs_hbm.at[sv], buf, sem)

    def gwait(buf, sem):
        pltpu.make_async_copy(hs_hbm.at[srca], buf, sem).wait()

    aload(src_hbm, 0, srca, isema)
    aload(dst_hbm, 0, dsta, dsema)
    aload(src_hbm, 1, srcb, isemb)
    aload(dst_hbm, 1, dstb, dsemb)
    await_(srca, isema)
    gstart(srca, rows_a, gsema)
    await_(srcb, isemb)
    gstart(srcb, rows_b, gsemb)

    def scstart(buf, dv, ssem):
        pltpu.async_copy(buf, agg_sh.at[dv], ssem, add=True)

    def scwait(buf, dv, ssem):
        pltpu.make_async_copy(buf, agg_sh.at[dv], ssem).wait()

    def step(i2, carry):
        c0 = 2 * i2
        gwait(rows_a, gsema)

        @pl.when(c0 + 2 < NCHUNK)
        def _():
            aload(src_hbm, c0 + 2, srca, isema)

        await_(dsta, dsema)
        pltpu.sync_copy(rows_a, agg_sh.at[dsta], add=True)

        @pl.when(c0 + 2 < NCHUNK)
        def _():
            aload(dst_hbm, c0 + 2, dsta, dsema)
            await_(srca, isema)
            gstart(srca, rows_a, gsema)

        gwait(rows_b, gsemb)

        @pl.when(c0 + 3 < NCHUNK)
        def _():
            aload(src_hbm, c0 + 3, srcb, isemb)

        await_(dstb, dsemb)
        pltpu.sync_copy(rows_b, agg_sh.at[dstb], add=True)

        @pl.when(c0 + 3 < NCHUNK)
        def _():
            aload(dst_hbm, c0 + 3, dstb, dsemb)
            await_(srcb, isemb)
            gstart(srcb, rows_b, gsemb)

        return carry

    lax.fori_loop(0, NCHUNK // 2, step, 0)
    plsc.subcore_barrier()
    for j in range(RPT // 128):
        r0 = s * RPT + j * 128
        pltpu.async_copy(agg_sh.at[pl.ds(r0, 128)],
                         out_hbm.at[c, pl.ds(r0, 128)], osem)
    for j in range(RPT // 128):
        r0 = s * RPT + j * 128
        pltpu.make_async_copy(agg_sh.at[pl.ds(r0, 128)],
                              out_hbm.at[c, pl.ds(r0, 128)], osem).wait()


_sc_agg = pl.kernel(
    _agg_body,
    out_type=jax.ShapeDtypeStruct((NC, NPAD, H), jnp.float32),
    mesh=_MESH,
    scratch_types=[
        pltpu.VMEM_SHARED((NPAD, H), jnp.float32),
        pltpu.VMEM((CH,), jnp.int32),
        pltpu.VMEM((CH,), jnp.int32),
        pltpu.VMEM((CH,), jnp.int32),
        pltpu.VMEM((CH,), jnp.int32),
        pltpu.VMEM((CH, H), jnp.float32),
        pltpu.VMEM((CH, H), jnp.float32),
        pltpu.SemaphoreType.DMA,
        pltpu.SemaphoreType.DMA,
        pltpu.SemaphoreType.DMA,
        pltpu.SemaphoreType.DMA,
        pltpu.SemaphoreType.DMA,
        pltpu.SemaphoreType.DMA,
        pltpu.SemaphoreType.DMA,
        pltpu.SemaphoreType.DMA,
        pltpu.SemaphoreType.DMA,
    ],
)


BLK = 2000
_NEG = -3.0e38


def _prep_body(x_ref, w_ref, deg_ref, gid_ref, hs_ref, ns_ref, nd_ref,
               cnt_ref):
    d = deg_ref[...]
    ns = lax.rsqrt(jnp.maximum(d[0, 0] + d[1, 0], 1.0))
    nd = lax.rsqrt(jnp.maximum(d[0, 1] + d[1, 1], 1.0))
    ns_ref[...] = ns
    nd_ref[...] = nd
    h = jnp.dot(x_ref[...], w_ref[...], preferred_element_type=jnp.float32)
    hs_ref[...] = h * ns
    mask = (gid_ref[...] == lax.broadcasted_iota(jnp.int32, (1, 16), 1))
    maskf = mask.astype(jnp.float32)
    cpart = lax.dot_general(maskf, jnp.ones((BLK, 1), jnp.float32),
                            (((0,), (0,)), ((), ())))

    @pl.when(pl.program_id(0) == 0)
    def _():
        cnt_ref[...] = jnp.zeros_like(cnt_ref)

    cnt_ref[...] += cpart


_tc_prep = pl.pallas_call(
    _prep_body,
    grid=(N // BLK,),
    in_specs=[
        pl.BlockSpec((BLK, F), lambda i: (i, 0)),
        pl.BlockSpec((F, H), lambda i: (0, 0)),
        pl.BlockSpec((2, 2, BLK, 1), lambda i: (0, 0, i, 0)),
        pl.BlockSpec((BLK, 1), lambda i: (i, 0)),
    ],
    out_specs=[
        pl.BlockSpec((BLK, H), lambda i: (i, 0)),
        pl.BlockSpec((BLK, 1), lambda i: (i, 0)),
        pl.BlockSpec((BLK, 1), lambda i: (i, 0)),
        pl.BlockSpec((16, 1), lambda i: (0, 0)),
    ],
    out_shape=[
        jax.ShapeDtypeStruct((N, H), jnp.float32),
        jax.ShapeDtypeStruct((N, 1), jnp.float32),
        jax.ShapeDtypeStruct((N, 1), jnp.float32),
        jax.ShapeDtypeStruct((16, 1), jnp.float32),
    ],
)


def _lupool(agg_ref, nd_ref, b_ref, p_ref):
    a = agg_ref[0] + agg_ref[1]
    g = jnp.maximum(a * nd_ref[...] + b_ref[...], 0.0)
    p = p_ref[...]
    scale = 1.0 / (jnp.sqrt(jnp.sum(p * p)) + 1e-8)
    score = lax.dot_general(g, p, (((1,), (1,)), ((), ())))
    return g * jnp.tanh(score * scale)


def _readout(xl, gid, av_ref, mx_ref):
    mask = (gid == lax.broadcasted_iota(jnp.int32, (1, 16), 1))
    maskf = mask.astype(jnp.float32)
    avp = lax.dot_general(maskf, xl, (((0,), (0,)), ((), ())))

    @pl.when(pl.program_id(0) == 0)
    def _():
        av_ref[...] = jnp.zeros_like(av_ref)
        mx_ref[...] = jnp.full_like(mx_ref, _NEG)

    av_ref[...] += avp
    rows = [jnp.max(jnp.where(mask[:, b:b + 1], xl, _NEG), axis=0,
                    keepdims=True) for b in range(B)]
    rows.append(jnp.full((16 - B, H), _NEG, jnp.float32))
    mx_ref[...] = jnp.maximum(mx_ref[...], jnp.concatenate(rows, axis=0))


def _post_body(agg_ref, nd_ref, b_ref, p_ref, w_ref, ns_ref, gid_ref,
               hs_ref, av_ref, mx_ref):
    xl = _lupool(agg_ref, nd_ref, b_ref, p_ref)
    hs_ref[...] = jnp.dot(xl, w_ref[...],
                          preferred_element_type=jnp.float32) * ns_ref[...]
    _readout(xl, gid_ref[...], av_ref, mx_ref)


_tc_post = pl.pallas_call(
    _post_body,
    grid=(N // BLK,),
    in_specs=[
        pl.BlockSpec((NC, BLK, H), lambda i: (0, i, 0)),
        pl.BlockSpec((BLK, 1), lambda i: (i, 0)),
        pl.BlockSpec((1, H), lambda i: (0, 0)),
        pl.BlockSpec((1, H), lambda i: (0, 0)),
        pl.BlockSpec((H, H), lambda i: (0, 0)),
        pl.BlockSpec((BLK, 1), lambda i: (i, 0)),
        pl.BlockSpec((BLK, 1), lambda i: (i, 0)),
    ],
    out_specs=[
        pl.BlockSpec((BLK, H), lambda i: (i, 0)),
        pl.BlockSpec((16, H), lambda i: (0, 0)),
        pl.BlockSpec((16, H), lambda i: (0, 0)),
    ],
    out_shape=[
        jax.ShapeDtypeStruct((N, H), jnp.float32),
        jax.ShapeDtypeStruct((16, H), jnp.float32),
        jax.ShapeDtypeStruct((16, H), jnp.float32),
    ],
)


def _post3_body(agg_ref, nd_ref, b_ref, p_ref, gid_ref, av_ref, mx_ref):
    xl = _lupool(agg_ref, nd_ref, b_ref, p_ref)
    _readout(xl, gid_ref[...], av_ref, mx_ref)


_tc_post3 = pl.pallas_call(
    _post3_body,
    grid=(N // BLK,),
    in_specs=[
        pl.BlockSpec((NC, BLK, H), lambda i: (0, i, 0)),
        pl.BlockSpec((BLK, 1), lambda i: (i, 0)),
        pl.BlockSpec((1, H), lambda i: (0, 0)),
        pl.BlockSpec((1, H), lambda i: (0, 0)),
        pl.BlockSpec((BLK, 1), lambda i: (i, 0)),
    ],
    out_specs=[
        pl.BlockSpec((16, H), lambda i: (0, 0)),
        pl.BlockSpec((16, H), lambda i: (0, 0)),
    ],
    out_shape=[
        jax.ShapeDtypeStruct((16, H), jnp.float32),
        jax.ShapeDtypeStruct((16, H), jnp.float32),
    ],
)


def _final_body(av1, av2, av3, mx1, mx2, mx3, cnt, xm1, x0, xp1, wc, cb,
                wtf, btf, wcat, bcat, wfc1, bfc1, wfc2, bfc2, wout, bout,
                w1, out_ref):
    counts = cnt[...]
    have = counts > 0.0
    c = jnp.maximum(counts, 1.0)
    av = (av1[...] + av2[...] + av3[...]) / c
    mx = (jnp.where(have, mx1[...], 0.0) + jnp.where(have, mx2[...], 0.0)
          + jnp.where(have, mx3[...], 0.0))
    rsum = jnp.concatenate([av, mx], axis=1)
    gnn = jnp.maximum(
        jnp.dot(rsum, wcat[...], preferred_element_type=jnp.float32)
        + bcat[...], 0.0)[0:B]

    conv = (jnp.dot(xm1[...], wc[0], preferred_element_type=jnp.float32)
            + jnp.dot(x0[...], wc[1], preferred_element_type=jnp.float32)
            + jnp.dot(xp1[...], wc[2], preferred_element_type=jnp.float32)
            + cb[...])
    conv = jnp.maximum(conv, 0.0)
    seq = jnp.max(conv.reshape(B, L, 128), axis=1)
    seq1 = jnp.maximum(
        jnp.dot(seq, wtf[...], preferred_element_type=jnp.float32)
        + btf[...], 0.0)

    s = jax.nn.sigmoid(w1[0, 0])
    gc1 = (1.0 - s) * gnn + s * seq1
    gc = jnp.maximum(
        jnp.dot(gc1, wfc1[...], preferred_element_type=jnp.float32)
        + bfc1[...], 0.0)
    gc = jnp.maximum(
        jnp.dot(gc, wfc2[...], preferred_element_type=jnp.float32)
        + bfc2[...], 0.0)
    o = jnp.maximum(
        jnp.dot(gc, wout[...], preferred_element_type=jnp.float32)
        + bout[...], 0.0)
    m = jnp.max(o, axis=1, keepdims=True)
    e = jnp.exp(o - m)
    out_ref[...] = e / jnp.sum(e, axis=1, keepdims=True)


_tc_final = pl.pallas_call(
    _final_body,
    out_shape=jax.ShapeDtypeStruct((B, 2), jnp.float32),
)


def kernel(x, edge_index, graph_ids, pad_dmap, Wg1, bg1, p1, Wg2, bg2, p2,
           Wg3, bg3, p3, conv_w, conv_b, Wtf, btf, Wcat, bcat, Wfc1, bfc1,
           Wfc2, bfc2, Wout, bout, w1):
    src = edge_index[0].reshape(NW, NCHUNK, CH)
    dst = edge_index[1].reshape(NW, NCHUNK, CH)
    gid2 = graph_ids.reshape(N, 1)

    deg = _sc_deg(src, dst).reshape(NC, 2, NPAD, 1)
    hs1, ns, nd, counts = _tc_prep(x, Wg1, deg, gid2)
    a1 = _sc_agg(hs1, src, dst)
    hs2, av1, mx1 = _tc_post(a1, nd, bg1.reshape(1, H), p1.reshape(1, H),
                             Wg2, ns, gid2)
    a2 = _sc_agg(hs2, src, dst)
    hs3, av2, mx2 = _tc_post(a2, nd, bg2.reshape(1, H), p2.reshape(1, H),
                             Wg3, ns, gid2)
    a3 = _sc_agg(hs3, src, dst)
    av3, mx3 = _tc_post3(a3, nd, bg3.reshape(1, H), p3.reshape(1, H), gid2)

    xt = jnp.transpose(pad_dmap, (0, 2, 1))
    xm1 = jnp.pad(xt, ((0, 0), (1, 0), (0, 0)))[:, :L].reshape(B * L, F)
    x0 = xt.reshape(B * L, F)
    xp1 = jnp.pad(xt, ((0, 0), (0, 1), (0, 0)))[:, 1:].reshape(B * L, F)
    wc = jnp.stack([conv_w[:, :, k].T for k in range(3)])

    return _tc_final(av1, av2, av3, mx1, mx2, mx3, counts, xm1, x0, xp1,
                     wc, conv_b.reshape(1, -1), Wtf, btf.reshape(1, -1),
                     Wcat, bcat.reshape(1, -1), Wfc1, bfc1.reshape(1, -1),
                     Wfc2, bfc2.reshape(1, -1), Wout, bout.reshape(1, -1),
                     w1.reshape(1, 1))


# speedup vs baseline: 1.3859x; 1.0606x over previous
import jax
import jax.numpy as jnp
from jax import lax
from jax.experimental import pallas as pl
from jax.experimental.pallas import tpu as pltpu
from jax.experimental.pallas import tpu_sc as plsc

N = 10000
E = 320000
F = 128
H = 128
B = 10
L = 200

NC, NS = 2, 16
NW = NC * NS
EPW = E // NW
CH = 125
NCHUNK = EPW // CH
NPAD = 10240
RPT = NPAD // NS

_MESH = plsc.VectorSubcoreMesh(
    core_axis_name="c", subcore_axis_name="s", num_cores=NC, num_subcores=NS)


def _deg_body(src_hbm, dst_hbm, out_hbm, dego_sh, degi_sh, src_v, dst_v,
              ones_v, zb_v, sema, semb):
    c = lax.axis_index("c")
    s = lax.axis_index("s")
    wid = c * NS + s

    pltpu.sync_copy(src_hbm.at[wid], src_v)
    pltpu.sync_copy(dst_hbm.at[wid], dst_v)

    def fill(i, carry):
        zb_v[pl.ds(i * 16, 16)] = jnp.zeros((16,), jnp.float32)
        return carry

    lax.fori_loop(0, RPT // 16, fill, 0)
    for j in range(128 // 16):
        ones_v[pl.ds(j * 16, 16)] = jnp.ones((16,), jnp.float32)

    pltpu.sync_copy(zb_v, dego_sh.at[pl.ds(s * RPT, RPT)])
    pltpu.sync_copy(zb_v, degi_sh.at[pl.ds(s * RPT, RPT)])
    plsc.subcore_barrier()

    ones = ones_v.at[pl.ds(0, CH)]

    def step(i, carry):
        pltpu.async_copy(ones, dego_sh.at[src_v.at[i]], sema, add=True)
        pltpu.async_copy(ones, degi_sh.at[dst_v.at[i]], semb, add=True)
        pltpu.make_async_copy(ones, dego_sh.at[src_v.at[i]], sema).wait()
        pltpu.make_async_copy(ones, degi_sh.at[dst_v.at[i]], semb).wait()
        return carry

    lax.fori_loop(0, NCHUNK, step, 0)
    plsc.subcore_barrier()
    pltpu.sync_copy(dego_sh.at[pl.ds(s * RPT, RPT)],
                    out_hbm.at[c, 0, pl.ds(s * RPT, RPT)])
    pltpu.sync_copy(degi_sh.at[pl.ds(s * RPT, RPT)],
                    out_hbm.at[c, 1, pl.ds(s * RPT, RPT)])


_sc_deg = pl.kernel(
    _deg_body,
    out_type=jax.ShapeDtypeStruct((NC, 2, NPAD), jnp.float32),
    mesh=_MESH,
    scratch_types=[
        pltpu.VMEM_SHARED((NPAD,), jnp.float32),
        pltpu.VMEM_SHARED((NPAD,), jnp.float32),
        pltpu.VMEM((NCHUNK, CH), jnp.int32),
        pltpu.VMEM((NCHUNK, CH), jnp.int32),
        pltpu.VMEM((128,), jnp.float32),
        pltpu.VMEM((RPT,), jnp.float32),
        pltpu.SemaphoreType.DMA,
        pltpu.SemaphoreType.DMA,
    ],
)


def _agg_body(hs_hbm, src_hbm, dst_hbm, out_hbm, agg_sh, srca, dsta, srcb,
              dstb, rows_a, rows_b, isema, isemb, dsema, dsemb, gsema,
              gsemb, ssema, ssemb, osem):
    c = lax.axis_index("c")
    s = lax.axis_index("s")
    wid = c * NS + s

    def fillrow(i, carry):
        for j in range(H // 16):
            rows_a[i, pl.ds(j * 16, 16)] = jnp.zeros((16,), jnp.float32)
        return carry

    lax.fori_loop(0, CH, fillrow, 0)
    nfull = RPT // CH
    rem = RPT - nfull * CH
    for j in range(nfull):
        pltpu.async_copy(rows_a, agg_sh.at[pl.ds(s * RPT + j * CH, CH)],
                         osem)
    pltpu.async_copy(rows_a.at[pl.ds(0, rem)],
                     agg_sh.at[pl.ds(s * RPT + nfull * CH, rem)], osem)
    for j in range(nfull):
        pltpu.make_async_copy(
            rows_a, agg_sh.at[pl.ds(s * RPT + j * CH, CH)], osem).wait()
    pltpu.make_async_copy(
        rows_a.at[pl.ds(0, rem)],
        agg_sh.at[pl.ds(s * RPT + nfull * CH, rem)], osem).wait()
    plsc.subcore_barrier()

    def aload(arr, i, buf, sem):
        pltpu.async_copy(arr.at[wid, i], buf, sem)

    def await_(buf, sem):
        pltpu.make_async_copy(src_hbm.at[wid, 0], buf, sem).wait()

    def gstart(sv, buf, sem):
        pltpu.async_copy(hs_hbm.at[sv], buf, sem)

    def gwait(buf, sem):
        pltpu.make_async_copy(hs_hbm.at[srca], buf, sem).wait()

    aload(src_hbm, 0, srca, isema)
    aload(dst_hbm, 0, dsta, dsema)
    aload(src_hbm, 1, srcb, isemb)
    aload(dst_hbm, 1, dstb, dsemb)
    await_(srca, isema)
    gstart(srca, rows_a, gsema)
    await_(srcb, isemb)
    gstart(srcb, rows_b, gsemb)

    def scstart(buf, dv, ssem):
        pltpu.async_copy(buf, agg_sh.at[dv], ssem, add=True)

    def scwait(buf, dv, ssem):
        pltpu.make_async_copy(buf, agg_sh.at[dv], ssem).wait()

    def step(i2, carry):
        c0 = 2 * i2
        gwait(rows_a, gsema)

        @pl.when(c0 + 2 < NCHUNK)
        def _():
            aload(src_hbm, c0 + 2, srca, isema)

        await_(dsta, dsema)
        pltpu.sync_copy(rows_a, agg_sh.at[dsta], add=True)

        @pl.when(c0 + 2 < NCHUNK)
        def _():
            aload(dst_hbm, c0 + 2, dsta, dsema)
            await_(srca, isema)
            gstart(srca, rows_a, gsema)

        gwait(rows_b, gsemb)

        @pl.when(c0 + 3 < NCHUNK)
        def _():
            aload(src_hbm, c0 + 3, srcb, isemb)

        await_(dstb, dsemb)
        pltpu.sync_copy(rows_b, agg_sh.at[dstb], add=True)

        @pl.when(c0 + 3 < NCHUNK)
        def _():
            aload(dst_hbm, c0 + 3, dstb, dsemb)
            await_(srcb, isemb)
            gstart(srcb, rows_b, gsemb)

        return carry

    lax.fori_loop(0, NCHUNK // 2, step, 0)
    plsc.subcore_barrier()
    for j in range(RPT // 128):
        r0 = s * RPT + j * 128
        pltpu.async_copy(agg_sh.at[pl.ds(r0, 128)],
                         out_hbm.at[c, pl.ds(r0, 128)], osem)
    for j in range(RPT // 128):
        r0 = s * RPT + j * 128
        pltpu.make_async_copy(agg_sh.at[pl.ds(r0, 128)],
                              out_hbm.at[c, pl.ds(r0, 128)], osem).wait()


_sc_agg = pl.kernel(
    _agg_body,
    out_type=jax.ShapeDtypeStruct((NC, NPAD, H), jnp.float32),
    mesh=_MESH,
    scratch_types=[
        pltpu.VMEM_SHARED((NPAD, H), jnp.float32),
        pltpu.VMEM((CH,), jnp.int32),
        pltpu.VMEM((CH,), jnp.int32),
        pltpu.VMEM((CH,), jnp.int32),
        pltpu.VMEM((CH,), jnp.int32),
        pltpu.VMEM((CH, H), jnp.float32),
        pltpu.VMEM((CH, H), jnp.float32),
        pltpu.SemaphoreType.DMA,
        pltpu.SemaphoreType.DMA,
        pltpu.SemaphoreType.DMA,
        pltpu.SemaphoreType.DMA,
        pltpu.SemaphoreType.DMA,
        pltpu.SemaphoreType.DMA,
        pltpu.SemaphoreType.DMA,
        pltpu.SemaphoreType.DMA,
        pltpu.SemaphoreType.DMA,
    ],
)


BLK = 2000
_NEG = -3.0e38


def _prep_body(x_ref, w_ref, deg_ref, gid_ref, hs_ref, ns_ref, nd_ref,
               cnt_ref):
    d = deg_ref[...]
    ns = lax.rsqrt(jnp.maximum(d[0, 0] + d[1, 0], 1.0))
    nd = lax.rsqrt(jnp.maximum(d[0, 1] + d[1, 1], 1.0))
    ns_ref[...] = ns
    nd_ref[...] = nd
    h = jnp.dot(x_ref[...], w_ref[...], preferred_element_type=jnp.float32)
    hs_ref[...] = h * ns
    mask = (gid_ref[...] == lax.broadcasted_iota(jnp.int32, (1, 16), 1))
    maskf = mask.astype(jnp.float32)
    cpart = lax.dot_general(maskf, jnp.ones((BLK, 1), jnp.float32),
                            (((0,), (0,)), ((), ())))

    @pl.when(pl.program_id(0) == 0)
    def _():
        cnt_ref[...] = jnp.zeros_like(cnt_ref)

    cnt_ref[...] += cpart


_tc_prep = pl.pallas_call(
    _prep_body,
    grid=(N // BLK,),
    in_specs=[
        pl.BlockSpec((BLK, F), lambda i: (i, 0)),
        pl.BlockSpec((F, H), lambda i: (0, 0)),
        pl.BlockSpec((2, 2, BLK, 1), lambda i: (0, 0, i, 0)),
        pl.BlockSpec((BLK, 1), lambda i: (i, 0)),
    ],
    out_specs=[
        pl.BlockSpec((BLK, H), lambda i: (i, 0)),
        pl.BlockSpec((BLK, 1), lambda i: (i, 0)),
        pl.BlockSpec((BLK, 1), lambda i: (i, 0)),
        pl.BlockSpec((16, 1), lambda i: (0, 0)),
    ],
    out_shape=[
        jax.ShapeDtypeStruct((N, H), jnp.float32),
        jax.ShapeDtypeStruct((N, 1), jnp.float32),
        jax.ShapeDtypeStruct((N, 1), jnp.float32),
        jax.ShapeDtypeStruct((16, 1), jnp.float32),
    ],
)


def _lupool(agg_ref, nd_ref, b_ref, p_ref):
    a = agg_ref[0] + agg_ref[1]
    g = jnp.maximum(a * nd_ref[...] + b_ref[...], 0.0)
    p = p_ref[...]
    scale = 1.0 / (jnp.sqrt(jnp.sum(p * p)) + 1e-8)
    score = lax.dot_general(g, p, (((1,), (1,)), ((), ())))
    return g * jnp.tanh(score * scale)


def _readout(xl, gid, av_ref, mx_ref):
    mask = (gid == lax.broadcasted_iota(jnp.int32, (1, 16), 1))
    maskf = mask.astype(jnp.float32)
    avp = lax.dot_general(maskf, xl, (((0,), (0,)), ((), ())))

    @pl.when(pl.program_id(0) == 0)
    def _():
        av_ref[...] = jnp.zeros_like(av_ref)
        mx_ref[...] = jnp.full_like(mx_ref, _NEG)

    av_ref[...] += avp
    g_lo = gid[0, 0]
    g_hi = gid[BLK - 1, 0]
    for b in range(B):
        @pl.when(jnp.logical_and(g_lo <= b, b <= g_hi))
        def _(b=b):
            red = jnp.max(jnp.where(mask[:, b:b + 1], xl, _NEG), axis=0,
                          keepdims=True)
            mx_ref[b:b + 1, :] = jnp.maximum(mx_ref[b:b + 1, :], red)


def _post_body(agg_ref, nd_ref, b_ref, p_ref, w_ref, ns_ref, gid_ref,
               hs_ref, av_ref, mx_ref):
    xl = _lupool(agg_ref, nd_ref, b_ref, p_ref)
    hs_ref[...] = jnp.dot(xl, w_ref[...],
                          preferred_element_type=jnp.float32) * ns_ref[...]
    _readout(xl, gid_ref[...], av_ref, mx_ref)


_tc_post = pl.pallas_call(
    _post_body,
    grid=(N // BLK,),
    in_specs=[
        pl.BlockSpec((NC, BLK, H), lambda i: (0, i, 0)),
        pl.BlockSpec((BLK, 1), lambda i: (i, 0)),
        pl.BlockSpec((1, H), lambda i: (0, 0)),
        pl.BlockSpec((1, H), lambda i: (0, 0)),
        pl.BlockSpec((H, H), lambda i: (0, 0)),
        pl.BlockSpec((BLK, 1), lambda i: (i, 0)),
        pl.BlockSpec((BLK, 1), lambda i: (i, 0)),
    ],
    out_specs=[
        pl.BlockSpec((BLK, H), lambda i: (i, 0)),
        pl.BlockSpec((16, H), lambda i: (0, 0)),
        pl.BlockSpec((16, H), lambda i: (0, 0)),
    ],
    out_shape=[
        jax.ShapeDtypeStruct((N, H), jnp.float32),
        jax.ShapeDtypeStruct((16, H), jnp.float32),
        jax.ShapeDtypeStruct((16, H), jnp.float32),
    ],
)


def _post3_body(agg_ref, nd_ref, b_ref, p_ref, gid_ref, av_ref, mx_ref):
    xl = _lupool(agg_ref, nd_ref, b_ref, p_ref)
    _readout(xl, gid_ref[...], av_ref, mx_ref)


_tc_post3 = pl.pallas_call(
    _post3_body,
    grid=(N // BLK,),
    in_specs=[
        pl.BlockSpec((NC, BLK, H), lambda i: (0, i, 0)),
        pl.BlockSpec((BLK, 1), lambda i: (i, 0)),
        pl.BlockSpec((1, H), lambda i: (0, 0)),
        pl.BlockSpec((1, H), lambda i: (0, 0)),
        pl.BlockSpec((BLK, 1), lambda i: (i, 0)),
    ],
    out_specs=[
        pl.BlockSpec((16, H), lambda i: (0, 0)),
        pl.BlockSpec((16, H), lambda i: (0, 0)),
    ],
    out_shape=[
        jax.ShapeDtypeStruct((16, H), jnp.float32),
        jax.ShapeDtypeStruct((16, H), jnp.float32),
    ],
)


def _final_body(av1, av2, av3, mx1, mx2, mx3, cnt, xm1, x0, xp1, wc, cb,
                wtf, btf, wcat, bcat, wfc1, bfc1, wfc2, bfc2, wout, bout,
                w1, out_ref):
    counts = cnt[...]
    have = counts > 0.0
    c = jnp.maximum(counts, 1.0)
    av = (av1[...] + av2[...] + av3[...]) / c
    mx = (jnp.where(have, mx1[...], 0.0) + jnp.where(have, mx2[...], 0.0)
          + jnp.where(have, mx3[...], 0.0))
    rsum = jnp.concatenate([av, mx], axis=1)
    gnn = jnp.maximum(
        jnp.dot(rsum, wcat[...], preferred_element_type=jnp.float32)
        + bcat[...], 0.0)[0:B]

    conv = (jnp.dot(xm1[...], wc[0], preferred_element_type=jnp.float32)
            + jnp.dot(x0[...], wc[1], preferred_element_type=jnp.float32)
            + jnp.dot(xp1[...], wc[2], preferred_element_type=jnp.float32)
            + cb[...])
    conv = jnp.maximum(conv, 0.0)
    seq = jnp.max(conv.reshape(B, L, 128), axis=1)
    seq1 = jnp.maximum(
        jnp.dot(seq, wtf[...], preferred_element_type=jnp.float32)
        + btf[...], 0.0)

    s = jax.nn.sigmoid(w1[0, 0])
    gc1 = (1.0 - s) * gnn + s * seq1
    gc = jnp.maximum(
        jnp.dot(gc1, wfc1[...], preferred_element_type=jnp.float32)
        + bfc1[...], 0.0)
    gc = jnp.maximum(
        jnp.dot(gc, wfc2[...], preferred_element_type=jnp.float32)
        + bfc2[...], 0.0)
    o = jnp.maximum(
        jnp.dot(gc, wout[...], preferred_element_type=jnp.float32)
        + bout[...], 0.0)
    m = jnp.max(o, axis=1, keepdims=True)
    e = jnp.exp(o - m)
    out_ref[...] = e / jnp.sum(e, axis=1, keepdims=True)


_tc_final = pl.pallas_call(
    _final_body,
    out_shape=jax.ShapeDtypeStruct((B, 2), jnp.float32),
)


def kernel(x, edge_index, graph_ids, pad_dmap, Wg1, bg1, p1, Wg2, bg2, p2,
           Wg3, bg3, p3, conv_w, conv_b, Wtf, btf, Wcat, bcat, Wfc1, bfc1,
           Wfc2, bfc2, Wout, bout, w1):
    src = edge_index[0].reshape(NW, NCHUNK, CH)
    dst = edge_index[1].reshape(NW, NCHUNK, CH)
    gid2 = graph_ids.reshape(N, 1)

    deg = _sc_deg(src, dst).reshape(NC, 2, NPAD, 1)
    hs1, ns, nd, counts = _tc_prep(x, Wg1, deg, gid2)
    a1 = _sc_agg(hs1, src, dst)
    hs2, av1, mx1 = _tc_post(a1, nd, bg1.reshape(1, H), p1.reshape(1, H),
                             Wg2, ns, gid2)
    a2 = _sc_agg(hs2, src, dst)
    hs3, av2, mx2 = _tc_post(a2, nd, bg2.reshape(1, H), p2.reshape(1, H),
                             Wg3, ns, gid2)
    a3 = _sc_agg(hs3, src, dst)
    av3, mx3 = _tc_post3(a3, nd, bg3.reshape(1, H), p3.reshape(1, H), gid2)

    xt = jnp.transpose(pad_dmap, (0, 2, 1))
    xm1 = jnp.pad(xt, ((0, 0), (1, 0), (0, 0)))[:, :L].reshape(B * L, F)
    x0 = xt.reshape(B * L, F)
    xp1 = jnp.pad(xt, ((0, 0), (0, 1), (0, 0)))[:, 1:].reshape(B * L, F)
    wc = jnp.stack([conv_w[:, :, k].T for k in range(3)])

    return _tc_final(av1, av2, av3, mx1, mx2, mx3, counts, xm1, x0, xp1,
                     wc, conv_b.reshape(1, -1), Wtf, btf.reshape(1, -1),
                     Wcat, bcat.reshape(1, -1), Wfc1, bfc1.reshape(1, -1),
                     Wfc2, bfc2.reshape(1, -1), Wout, bout.reshape(1, -1),
                     w1.reshape(1, 1))


# speedup vs baseline: 1.3973x; 1.0083x over previous
import jax
import jax.numpy as jnp
from jax import lax
from jax.experimental import pallas as pl
from jax.experimental.pallas import tpu as pltpu
from jax.experimental.pallas import tpu_sc as plsc

N = 10000
E = 320000
F = 128
H = 128
B = 10
L = 200

NC, NS = 2, 16
NW = NC * NS
EPW = E // NW
CH = 125
NCHUNK = EPW // CH
NPAD = 10240
RPT = NPAD // NS

_MESH = plsc.VectorSubcoreMesh(
    core_axis_name="c", subcore_axis_name="s", num_cores=NC, num_subcores=NS)


def _deg_body(src_hbm, dst_hbm, out_hbm, dego_sh, degi_sh, src_v, dst_v,
              ones_v, zb_v, sema, semb):
    c = lax.axis_index("c")
    s = lax.axis_index("s")
    wid = c * NS + s

    pltpu.sync_copy(src_hbm.at[wid], src_v)
    pltpu.sync_copy(dst_hbm.at[wid], dst_v)

    def fill(i, carry):
        zb_v[pl.ds(i * 16, 16)] = jnp.zeros((16,), jnp.float32)
        return carry

    lax.fori_loop(0, RPT // 16, fill, 0)
    for j in range(128 // 16):
        ones_v[pl.ds(j * 16, 16)] = jnp.ones((16,), jnp.float32)

    pltpu.sync_copy(zb_v, dego_sh.at[pl.ds(s * RPT, RPT)])
    pltpu.sync_copy(zb_v, degi_sh.at[pl.ds(s * RPT, RPT)])
    plsc.subcore_barrier()

    ones = ones_v.at[pl.ds(0, CH)]

    def step(i, carry):
        pltpu.async_copy(ones, dego_sh.at[src_v.at[i]], sema, add=True)
        pltpu.async_copy(ones, degi_sh.at[dst_v.at[i]], semb, add=True)

        @pl.when(i > 0)
        def _():
            pltpu.make_async_copy(ones, dego_sh.at[src_v.at[i]],
                                  sema).wait()
            pltpu.make_async_copy(ones, degi_sh.at[dst_v.at[i]],
                                  semb).wait()

        return carry

    lax.fori_loop(0, NCHUNK, step, 0)
    pltpu.make_async_copy(ones, dego_sh.at[src_v.at[0]], sema).wait()
    pltpu.make_async_copy(ones, degi_sh.at[dst_v.at[0]], semb).wait()
    plsc.subcore_barrier()
    pltpu.sync_copy(dego_sh.at[pl.ds(s * RPT, RPT)],
                    out_hbm.at[c, 0, pl.ds(s * RPT, RPT)])
    pltpu.sync_copy(degi_sh.at[pl.ds(s * RPT, RPT)],
                    out_hbm.at[c, 1, pl.ds(s * RPT, RPT)])


_sc_deg = pl.kernel(
    _deg_body,
    out_type=jax.ShapeDtypeStruct((NC, 2, NPAD), jnp.float32),
    mesh=_MESH,
    scratch_types=[
        pltpu.VMEM_SHARED((NPAD,), jnp.float32),
        pltpu.VMEM_SHARED((NPAD,), jnp.float32),
        pltpu.VMEM((NCHUNK, CH), jnp.int32),
        pltpu.VMEM((NCHUNK, CH), jnp.int32),
        pltpu.VMEM((128,), jnp.float32),
        pltpu.VMEM((RPT,), jnp.float32),
        pltpu.SemaphoreType.DMA,
        pltpu.SemaphoreType.DMA,
    ],
)


def _agg_body(hs_hbm, src_hbm, dst_hbm, out_hbm, agg_sh, srca, dsta, srcb,
              dstb, rows_a, rows_b, isema, isemb, dsema, dsemb, gsema,
              gsemb, ssema, ssemb, osem):
    c = lax.axis_index("c")
    s = lax.axis_index("s")
    wid = c * NS + s

    def fillrow(i, carry):
        for j in range(H // 16):
            rows_a[i, pl.ds(j * 16, 16)] = jnp.zeros((16,), jnp.float32)
        return carry

    lax.fori_loop(0, CH, fillrow, 0)
    nfull = RPT // CH
    rem = RPT - nfull * CH
    for j in range(nfull):
        pltpu.async_copy(rows_a, agg_sh.at[pl.ds(s * RPT + j * CH, CH)],
                         osem)
    pltpu.async_copy(rows_a.at[pl.ds(0, rem)],
                     agg_sh.at[pl.ds(s * RPT + nfull * CH, rem)], osem)
    for j in range(nfull):
        pltpu.make_async_copy(
            rows_a, agg_sh.at[pl.ds(s * RPT + j * CH, CH)], osem).wait()
    pltpu.make_async_copy(
        rows_a.at[pl.ds(0, rem)],
        agg_sh.at[pl.ds(s * RPT + nfull * CH, rem)], osem).wait()
    plsc.subcore_barrier()

    def aload(arr, i, buf, sem):
        pltpu.async_copy(arr.at[wid, i], buf, sem)

    def await_(buf, sem):
        pltpu.make_async_copy(src_hbm.at[wid, 0], buf, sem).wait()

    def gstart(sv, buf, sem):
        pltpu.async_copy(hs_hbm.at[sv], buf, sem)

    def gwait(buf, sem):
        pltpu.make_async_copy(hs_hbm.at[srca], buf, sem).wait()

    aload(src_hbm, 0, srca, isema)
    aload(dst_hbm, 0, dsta, dsema)
    aload(src_hbm, 1, srcb, isemb)
    aload(dst_hbm, 1, dstb, dsemb)
    await_(srca, isema)
    gstart(srca, rows_a, gsema)
    await_(srcb, isemb)
    gstart(srcb, rows_b, gsemb)

    def scstart(buf, dv, ssem):
        pltpu.async_copy(buf, agg_sh.at[dv], ssem, add=True)

    def scwait(buf, dv, ssem):
        pltpu.make_async_copy(buf, agg_sh.at[dv], ssem).wait()

    def step(i2, carry):
        c0 = 2 * i2
        gwait(rows_a, gsema)

        @pl.when(c0 + 2 < NCHUNK)
        def _():
            aload(src_hbm, c0 + 2, srca, isema)

        await_(dsta, dsema)
        pltpu.sync_copy(rows_a, agg_sh.at[dsta], add=True)

        @pl.when(c0 + 2 < NCHUNK)
        def _():
            aload(dst_hbm, c0 + 2, dsta, dsema)
            await_(srca, isema)
            gstart(srca, rows_a, gsema)

        gwait(rows_b, gsemb)

        @pl.when(c0 + 3 < NCHUNK)
        def _():
            aload(src_hbm, c0 + 3, srcb, isemb)

        await_(dstb, dsemb)
        pltpu.sync_copy(rows_b, agg_sh.at[dstb], add=True)

        @pl.when(c0 + 3 < NCHUNK)
        def _():
            aload(dst_hbm, c0 + 3, dstb, dsemb)
            await_(srcb, isemb)
            gstart(srcb, rows_b, gsemb)

        return carry

    lax.fori_loop(0, NCHUNK // 2, step, 0)
    plsc.subcore_barrier()
    for j in range(RPT // 128):
        r0 = s * RPT + j * 128
        pltpu.async_copy(agg_sh.at[pl.ds(r0, 128)],
                         out_hbm.at[c, pl.ds(r0, 128)], osem)
    for j in range(RPT // 128):
        r0 = s * RPT + j * 128
        pltpu.make_async_copy(agg_sh.at[pl.ds(r0, 128)],
                              out_hbm.at[c, pl.ds(r0, 128)], osem).wait()


_sc_agg = pl.kernel(
    _agg_body,
    out_type=jax.ShapeDtypeStruct((NC, NPAD, H), jnp.float32),
    mesh=_MESH,
    scratch_types=[
        pltpu.VMEM_SHARED((NPAD, H), jnp.float32),
        pltpu.VMEM((CH,), jnp.int32),
        pltpu.VMEM((CH,), jnp.int32),
        pltpu.VMEM((CH,), jnp.int32),
        pltpu.VMEM((CH,), jnp.int32),
        pltpu.VMEM((CH, H), jnp.float32),
        pltpu.VMEM((CH, H), jnp.float32),
        pltpu.SemaphoreType.DMA,
        pltpu.SemaphoreType.DMA,
        pltpu.SemaphoreType.DMA,
        pltpu.SemaphoreType.DMA,
        pltpu.SemaphoreType.DMA,
        pltpu.SemaphoreType.DMA,
        pltpu.SemaphoreType.DMA,
        pltpu.SemaphoreType.DMA,
        pltpu.SemaphoreType.DMA,
    ],
)


BLK = 2000
_NEG = -3.0e38


def _prep_body(x_ref, w_ref, deg_ref, gid_ref, hs_ref, ns_ref, nd_ref,
               cnt_ref):
    d = deg_ref[...]
    ns = lax.rsqrt(jnp.maximum(d[0, 0] + d[1, 0], 1.0))
    nd = lax.rsqrt(jnp.maximum(d[0, 1] + d[1, 1], 1.0))
    ns_ref[...] = ns
    nd_ref[...] = nd
    h = jnp.dot(x_ref[...], w_ref[...], preferred_element_type=jnp.float32)
    hs_ref[...] = h * ns
    mask = (gid_ref[...] == lax.broadcasted_iota(jnp.int32, (1, 16), 1))
    maskf = mask.astype(jnp.float32)
    cpart = lax.dot_general(maskf, jnp.ones((BLK, 1), jnp.float32),
                            (((0,), (0,)), ((), ())))

    @pl.when(pl.program_id(0) == 0)
    def _():
        cnt_ref[...] = jnp.zeros_like(cnt_ref)

    cnt_ref[...] += cpart


_tc_prep = pl.pallas_call(
    _prep_body,
    grid=(N // BLK,),
    in_specs=[
        pl.BlockSpec((BLK, F), lambda i: (i, 0)),
        pl.BlockSpec((F, H), lambda i: (0, 0)),
        pl.BlockSpec((2, 2, BLK, 1), lambda i: (0, 0, i, 0)),
        pl.BlockSpec((BLK, 1), lambda i: (i, 0)),
    ],
    out_specs=[
        pl.BlockSpec((BLK, H), lambda i: (i, 0)),
        pl.BlockSpec((BLK, 1), lambda i: (i, 0)),
        pl.BlockSpec((BLK, 1), lambda i: (i, 0)),
        pl.BlockSpec((16, 1), lambda i: (0, 0)),
    ],
    out_shape=[
        jax.ShapeDtypeStruct((N, H), jnp.float32),
        jax.ShapeDtypeStruct((N, 1), jnp.float32),
        jax.ShapeDtypeStruct((N, 1), jnp.float32),
        jax.ShapeDtypeStruct((16, 1), jnp.float32),
    ],
)


def _lupool(agg_ref, nd_ref, b_ref, p_ref):
    a = agg_ref[0] + agg_ref[1]
    g = jnp.maximum(a * nd_ref[...] + b_ref[...], 0.0)
    p = p_ref[...]
    scale = 1.0 / (jnp.sqrt(jnp.sum(p * p)) + 1e-8)
    score = lax.dot_general(g, p, (((1,), (1,)), ((), ())))
    return g * jnp.tanh(score * scale)


def _readout(xl, gid, av_ref, mx_ref):
    mask = (gid == lax.broadcasted_iota(jnp.int32, (1, 16), 1))
    maskf = mask.astype(jnp.float32)
    avp = lax.dot_general(maskf, xl, (((0,), (0,)), ((), ())))

    @pl.when(pl.program_id(0) == 0)
    def _():
        av_ref[...] = jnp.zeros_like(av_ref)
        mx_ref[...] = jnp.full_like(mx_ref, _NEG)

    av_ref[...] += avp
    g_lo = gid[0, 0]
    g_hi = gid[BLK - 1, 0]
    for b in range(B):
        @pl.when(jnp.logical_and(g_lo <= b, b <= g_hi))
        def _(b=b):
            red = jnp.max(jnp.where(mask[:, b:b + 1], xl, _NEG), axis=0,
                          keepdims=True)
            mx_ref[b:b + 1, :] = jnp.maximum(mx_ref[b:b + 1, :], red)


def _post_body(agg_ref, nd_ref, b_ref, p_ref, w_ref, ns_ref, gid_ref,
               hs_ref, av_ref, mx_ref):
    xl = _lupool(agg_ref, nd_ref, b_ref, p_ref)
    hs_ref[...] = jnp.dot(xl, w_ref[...],
                          preferred_element_type=jnp.float32) * ns_ref[...]
    _readout(xl, gid_ref[...], av_ref, mx_ref)


_tc_post = pl.pallas_call(
    _post_body,
    grid=(N // BLK,),
    in_specs=[
        pl.BlockSpec((NC, BLK, H), lambda i: (0, i, 0)),
        pl.BlockSpec((BLK, 1), lambda i: (i, 0)),
        pl.BlockSpec((1, H), lambda i: (0, 0)),
        pl.BlockSpec((1, H), lambda i: (0, 0)),
        pl.BlockSpec((H, H), lambda i: (0, 0)),
        pl.BlockSpec((BLK, 1), lambda i: (i, 0)),
        pl.BlockSpec((BLK, 1), lambda i: (i, 0)),
    ],
    out_specs=[
        pl.BlockSpec((BLK, H), lambda i: (i, 0)),
        pl.BlockSpec((16, H), lambda i: (0, 0)),
        pl.BlockSpec((16, H), lambda i: (0, 0)),
    ],
    out_shape=[
        jax.ShapeDtypeStruct((N, H), jnp.float32),
        jax.ShapeDtypeStruct((16, H), jnp.float32),
        jax.ShapeDtypeStruct((16, H), jnp.float32),
    ],
)


def _post3_body(agg_ref, nd_ref, b_ref, p_ref, gid_ref, av_ref, mx_ref):
    xl = _lupool(agg_ref, nd_ref, b_ref, p_ref)
    _readout(xl, gid_ref[...], av_ref, mx_ref)


_tc_post3 = pl.pallas_call(
    _post3_body,
    grid=(N // BLK,),
    in_specs=[
        pl.BlockSpec((NC, BLK, H), lambda i: (0, i, 0)),
        pl.BlockSpec((BLK, 1), lambda i: (i, 0)),
        pl.BlockSpec((1, H), lambda i: (0, 0)),
        pl.BlockSpec((1, H), lambda i: (0, 0)),
        pl.BlockSpec((BLK, 1), lambda i: (i, 0)),
    ],
    out_specs=[
        pl.BlockSpec((16, H), lambda i: (0, 0)),
        pl.BlockSpec((16, H), lambda i: (0, 0)),
    ],
    out_shape=[
        jax.ShapeDtypeStruct((16, H), jnp.float32),
        jax.ShapeDtypeStruct((16, H), jnp.float32),
    ],
)


def _final_body(av1, av2, av3, mx1, mx2, mx3, cnt, xm1, x0, xp1, wc, cb,
                wtf, btf, wcat, bcat, wfc1, bfc1, wfc2, bfc2, wout, bout,
                w1, out_ref):
    counts = cnt[...]
    have = counts > 0.0
    c = jnp.maximum(counts, 1.0)
    av = (av1[...] + av2[...] + av3[...]) / c
    mx = (jnp.where(have, mx1[...], 0.0) + jnp.where(have, mx2[...], 0.0)
          + jnp.where(have, mx3[...], 0.0))
    rsum = jnp.concatenate([av, mx], axis=1)
    gnn = jnp.maximum(
        jnp.dot(rsum, wcat[...], preferred_element_type=jnp.float32)
        + bcat[...], 0.0)[0:B]

    conv = (jnp.dot(xm1[...], wc[0], preferred_element_type=jnp.float32)
            + jnp.dot(x0[...], wc[1], preferred_element_type=jnp.float32)
            + jnp.dot(xp1[...], wc[2], preferred_element_type=jnp.float32)
            + cb[...])
    conv = jnp.maximum(conv, 0.0)
    seq = jnp.max(conv.reshape(B, L, 128), axis=1)
    seq1 = jnp.maximum(
        jnp.dot(seq, wtf[...], preferred_element_type=jnp.float32)
        + btf[...], 0.0)

    s = jax.nn.sigmoid(w1[0, 0])
    gc1 = (1.0 - s) * gnn + s * seq1
    gc = jnp.maximum(
        jnp.dot(gc1, wfc1[...], preferred_element_type=jnp.float32)
        + bfc1[...], 0.0)
    gc = jnp.maximum(
        jnp.dot(gc, wfc2[...], preferred_element_type=jnp.float32)
        + bfc2[...], 0.0)
    o = jnp.maximum(
        jnp.dot(gc, wout[...], preferred_element_type=jnp.float32)
        + bout[...], 0.0)
    m = jnp.max(o, axis=1, keepdims=True)
    e = jnp.exp(o - m)
    out_ref[...] = e / jnp.sum(e, axis=1, keepdims=True)


_tc_final = pl.pallas_call(
    _final_body,
    out_shape=jax.ShapeDtypeStruct((B, 2), jnp.float32),
)


def kernel(x, edge_index, graph_ids, pad_dmap, Wg1, bg1, p1, Wg2, bg2, p2,
           Wg3, bg3, p3, conv_w, conv_b, Wtf, btf, Wcat, bcat, Wfc1, bfc1,
           Wfc2, bfc2, Wout, bout, w1):
    src = edge_index[0].reshape(NW, NCHUNK, CH)
    dst = edge_index[1].reshape(NW, NCHUNK, CH)
    gid2 = graph_ids.reshape(N, 1)

    deg = _sc_deg(src, dst).reshape(NC, 2, NPAD, 1)
    hs1, ns, nd, counts = _tc_prep(x, Wg1, deg, gid2)
    a1 = _sc_agg(hs1, src, dst)
    hs2, av1, mx1 = _tc_post(a1, nd, bg1.reshape(1, H), p1.reshape(1, H),
                             Wg2, ns, gid2)
    a2 = _sc_agg(hs2, src, dst)
    hs3, av2, mx2 = _tc_post(a2, nd, bg2.reshape(1, H), p2.reshape(1, H),
                             Wg3, ns, gid2)
    a3 = _sc_agg(hs3, src, dst)
    av3, mx3 = _tc_post3(a3, nd, bg3.reshape(1, H), p3.reshape(1, H), gid2)

    xt = jnp.transpose(pad_dmap, (0, 2, 1))
    xm1 = jnp.pad(xt, ((0, 0), (1, 0), (0, 0)))[:, :L].reshape(B * L, F)
    x0 = xt.reshape(B * L, F)
    xp1 = jnp.pad(xt, ((0, 0), (0, 1), (0, 0)))[:, 1:].reshape(B * L, F)
    wc = jnp.stack([conv_w[:, :, k].T for k in range(3)])

    return _tc_final(av1, av2, av3, mx1, mx2, mx3, counts, xm1, x0, xp1,
                     wc, conv_b.reshape(1, -1), Wtf, btf.reshape(1, -1),
                     Wcat, bcat.reshape(1, -1), Wfc1, bfc1.reshape(1, -1),
                     Wfc2, bfc2.reshape(1, -1), Wout, bout.reshape(1, -1),
                     w1.reshape(1, 1))
